# baseline (speedup 1.0000x reference)
import functools

import jax
import jax.numpy as jnp
from jax import lax
from jax.experimental import pallas as pl
from jax.experimental.pallas import tpu as pltpu
from jax.experimental.pallas import tpu_sc as plsc

P = 8192
F = 8192
V = 4096
BP = 256
BF = 512
RS = 8
NI = P // BP
NJ = F // BF

def _safe_div(num, den):
    den_s = jnp.where(jnp.abs(den) > 1e-12, den, jnp.ones_like(den))
    return num / den_s


def _tc_body(pts_ref, ftr_ref, idx_ref, w1_ref, w2_ref, w3_ref,
             rd_ref, ri_ref, ru_ref, rv_ref, rw_ref):
    j = pl.program_id(1)

    gid = j * BF + lax.broadcasted_iota(jnp.int32, (RS, BF), 1)
    first = j == 0

    def strip(r, carry):
        sl = pl.ds(r * RS, RS)
        px = pts_ref[sl, 0:1]
        py = pts_ref[sl, 1:2]
        pz = pts_ref[sl, 2:3]

        ax = ftr_ref[0:1, :]
        ay = ftr_ref[1:2, :]
        az = ftr_ref[2:3, :]
        bx = ftr_ref[3:4, :]
        by = ftr_ref[4:5, :]
        bz = ftr_ref[5:6, :]
        cx = ftr_ref[6:7, :]
        cy = ftr_ref[7:8, :]
        cz = ftr_ref[8:9, :]

        abx = bx - ax
        aby = by - ay
        abz = bz - az
        acx = cx - ax
        acy = cy - ay
        acz = cz - az

        apx = px - ax
        apy = py - ay
        apz = pz - az
        d1 = abx * apx + aby * apy + abz * apz
        d2 = acx * apx + acy * apy + acz * apz
        bpx = px - bx
        bpy = py - by
        bpz = pz - bz
        d3 = abx * bpx + aby * bpy + abz * bpz
        d4 = acx * bpx + acy * bpy + acz * bpz
        cpx = px - cx
        cpy = py - cy
        cpz = pz - cz
        d5 = abx * cpx + aby * cpy + abz * cpz
        d6 = acx * cpx + acy * cpy + acz * cpz

        vc = d1 * d4 - d3 * d2
        vb = d5 * d2 - d1 * d6
        va = d3 * d6 - d5 * d4
        v_ab = _safe_div(d1, d1 - d3)
        w_ac = _safe_div(d2, d2 - d6)
        w_bc = _safe_div(d4 - d3, (d4 - d3) + (d5 - d6))
        denom = _safe_div(jnp.ones_like(va), va + vb + vc)
        v_in = vb * denom
        w_in = vc * denom
        cond_a = (d1 <= 0.0) & (d2 <= 0.0)
        cond_b = (d3 >= 0.0) & (d4 <= d3)
        cond_ab = (vc <= 0.0) & (d1 >= 0.0) & (d3 <= 0.0)
        cond_c = (d6 >= 0.0) & (d5 <= d6)
        cond_ac = (vb <= 0.0) & (d2 >= 0.0) & (d6 <= 0.0)
        cond_bc = (va <= 0.0) & ((d4 - d3) >= 0.0) & ((d5 - d6) >= 0.0)

        u = 1.0 - v_in - w_in
        v = v_in
        w = w_in
        u = jnp.where(cond_bc, 0.0, u)
        v = jnp.where(cond_bc, 1.0 - w_bc, v)
        w = jnp.where(cond_bc, w_bc, w)
        u = jnp.where(cond_ac, 1.0 - w_ac, u)
        v = jnp.where(cond_ac, 0.0, v)
        w = jnp.where(cond_ac, w_ac, w)
        u = jnp.where(cond_c, 0.0, u)
        v = jnp.where(cond_c, 0.0, v)
        w = jnp.where(cond_c, 1.0, w)
        u = jnp.where(cond_ab, 1.0 - v_ab, u)
        v = jnp.where(cond_ab, v_ab, v)
        w = jnp.where(cond_ab, 0.0, w)
        u = jnp.where(cond_b, 0.0, u)
        v = jnp.where(cond_b, 1.0, v)
        w = jnp.where(cond_b, 0.0, w)
        u = jnp.where(cond_a, 1.0, u)
        v = jnp.where(cond_a, 0.0, v)
        w = jnp.where(cond_a, 0.0, w)

        qx = u * ax + v * bx + w * cx
        qy = u * ay + v * by + w * cy
        qz = u * az + v * bz + w * cz
        dx = px - qx
        dy = py - qy
        dz = pz - qz
        dist = dx * dx + dy * dy + dz * dz

        prev_d = rd_ref[sl, :]
        better = first | (dist < prev_d)
        rd_ref[sl, :] = jnp.where(better, dist, prev_d)
        ri_ref[sl, :] = jnp.where(better, gid, ri_ref[sl, :])
        ru_ref[sl, :] = jnp.where(better, u, ru_ref[sl, :])
        rv_ref[sl, :] = jnp.where(better, v, rv_ref[sl, :])
        rw_ref[sl, :] = jnp.where(better, w, rw_ref[sl, :])
        return carry

    lax.fori_loop(0, BP // RS, strip, 0, unroll=32)

    @pl.when(j == NJ - 1)
    def _():
        rd = rd_ref[...]
        ri = ri_ref[...]
        m = jnp.min(rd, axis=1, keepdims=True)
        ismin = rd == m
        cand = jnp.where(ismin, ri, 2 ** 30)
        sel = jnp.min(cand, axis=1)
        selmask = ismin & (ri == sel[:, None])
        idx_ref[...] = sel
        w1_ref[...] = jnp.sum(jnp.where(selmask, ru_ref[...], 0.0), axis=1)
        w2_ref[...] = jnp.sum(jnp.where(selmask, rv_ref[...], 0.0), axis=1)
        w3_ref[...] = jnp.sum(jnp.where(selmask, rw_ref[...], 0.0), axis=1)


def _pairwise_tc(points, ftr, interpret=False):
    out_shapes = (
        jax.ShapeDtypeStruct((P,), jnp.int32),
        jax.ShapeDtypeStruct((P,), jnp.float32),
        jax.ShapeDtypeStruct((P,), jnp.float32),
        jax.ShapeDtypeStruct((P,), jnp.float32),
    )
    return pl.pallas_call(
        _tc_body,
        grid=(NI, NJ),
        in_specs=[
            pl.BlockSpec((BP, 3), lambda i, j: (i, 0)),
            pl.BlockSpec((9, BF), lambda i, j: (0, j)),
        ],
        out_specs=(
            pl.BlockSpec((BP,), lambda i, j: (i,)),
            pl.BlockSpec((BP,), lambda i, j: (i,)),
            pl.BlockSpec((BP,), lambda i, j: (i,)),
            pl.BlockSpec((BP,), lambda i, j: (i,)),
        ),
        out_shape=out_shapes,
        scratch_shapes=[
            pltpu.VMEM((BP, BF), jnp.float32),
            pltpu.VMEM((BP, BF), jnp.int32),
            pltpu.VMEM((BP, BF), jnp.float32),
            pltpu.VMEM((BP, BF), jnp.float32),
            pltpu.VMEM((BP, BF), jnp.float32),
        ],
        compiler_params=pltpu.CompilerParams(
            dimension_semantics=("parallel", "arbitrary"),
        ),
        interpret=interpret,
    )(points, ftr)


_NW = 32
_BG = 3 * F
_BPW = _BG // _NW


def _sc_gather(vert_pad, idx_flat):
    mesh = plsc.VectorSubcoreMesh(core_axis_name="c", subcore_axis_name="s")

    @functools.partial(
        pl.kernel,
        mesh=mesh,
        out_type=jax.ShapeDtypeStruct((_BG, 128), jnp.float32),
        scratch_types=[
            pltpu.VMEM((_BPW,), jnp.int32),
            pltpu.VMEM((_BPW, 128), jnp.float32),
            pltpu.SemaphoreType.DMA,
        ],
    )
    def gk(table_hbm, idx_hbm, out_hbm, idx_v, rows_v, sem):
        wid = lax.axis_index("s") * 2 + lax.axis_index("c")
        base = wid * _BPW
        pltpu.sync_copy(idx_hbm.at[pl.ds(base, _BPW)], idx_v)
        pltpu.async_copy(table_hbm.at[idx_v], rows_v, sem).wait()
        pltpu.sync_copy(rows_v, out_hbm.at[pl.ds(base, _BPW)])

    return gk(vert_pad, idx_flat)


def kernel(points, vertices, faces):
    vert_pad = jnp.pad(vertices, ((0, 0), (0, 125)))
    idx_flat = faces.T.reshape(-1)
    g = _sc_gather(vert_pad, idx_flat)
    ftr = g.reshape(3, F, 128)[:, :, :3].transpose(0, 2, 1).reshape(9, F)
    return _pairwise_tc(points, ftr)


# speedup vs baseline: 1.0601x; 1.0601x over previous
import functools

import jax
import jax.numpy as jnp
from jax import lax
from jax.experimental import pallas as pl
from jax.experimental.pallas import tpu as pltpu
from jax.experimental.pallas import tpu_sc as plsc

P = 8192
F = 8192
V = 4096
BP = 256
BF = 256
RS = 8
NI = P // BP
NJ = F // BF
BW = 1024


def _safe_div(num, den):
    den_s = jnp.where(jnp.abs(den) > 1e-12, den, jnp.ones_like(den))
    return num / den_s


def _closest(px, py, pz, ax, ay, az, bx, by, bz, cx, cy, cz):
    abx = bx - ax
    aby = by - ay
    abz = bz - az
    acx = cx - ax
    acy = cy - ay
    acz = cz - az

    apx = px - ax
    apy = py - ay
    apz = pz - az
    d1 = abx * apx + aby * apy + abz * apz
    d2 = acx * apx + acy * apy + acz * apz
    bpx = px - bx
    bpy = py - by
    bpz = pz - bz
    d3 = abx * bpx + aby * bpy + abz * bpz
    d4 = acx * bpx + acy * bpy + acz * bpz
    cpx = px - cx
    cpy = py - cy
    cpz = pz - cz
    d5 = abx * cpx + aby * cpy + abz * cpz
    d6 = acx * cpx + acy * cpy + acz * cpz

    vc = d1 * d4 - d3 * d2
    vb = d5 * d2 - d1 * d6
    va = d3 * d6 - d5 * d4
    v_ab = _safe_div(d1, d1 - d3)
    w_ac = _safe_div(d2, d2 - d6)
    w_bc = _safe_div(d4 - d3, (d4 - d3) + (d5 - d6))
    denom = _safe_div(jnp.ones_like(va), va + vb + vc)
    v_in = vb * denom
    w_in = vc * denom
    cond_a = (d1 <= 0.0) & (d2 <= 0.0)
    cond_b = (d3 >= 0.0) & (d4 <= d3)
    cond_ab = (vc <= 0.0) & (d1 >= 0.0) & (d3 <= 0.0)
    cond_c = (d6 >= 0.0) & (d5 <= d6)
    cond_ac = (vb <= 0.0) & (d2 >= 0.0) & (d6 <= 0.0)
    cond_bc = (va <= 0.0) & ((d4 - d3) >= 0.0) & ((d5 - d6) >= 0.0)

    u = 1.0 - v_in - w_in
    v = v_in
    w = w_in
    u = jnp.where(cond_bc, 0.0, u)
    v = jnp.where(cond_bc, 1.0 - w_bc, v)
    w = jnp.where(cond_bc, w_bc, w)
    u = jnp.where(cond_ac, 1.0 - w_ac, u)
    v = jnp.where(cond_ac, 0.0, v)
    w = jnp.where(cond_ac, w_ac, w)
    u = jnp.where(cond_c, 0.0, u)
    v = jnp.where(cond_c, 0.0, v)
    w = jnp.where(cond_c, 1.0, w)
    u = jnp.where(cond_ab, 1.0 - v_ab, u)
    v = jnp.where(cond_ab, v_ab, v)
    w = jnp.where(cond_ab, 0.0, w)
    u = jnp.where(cond_b, 0.0, u)
    v = jnp.where(cond_b, 1.0, v)
    w = jnp.where(cond_b, 0.0, w)
    u = jnp.where(cond_a, 1.0, u)
    v = jnp.where(cond_a, 0.0, v)
    w = jnp.where(cond_a, 0.0, w)

    qx = u * ax + v * bx + w * cx
    qy = u * ay + v * by + w * cy
    qz = u * az + v * bz + w * cz
    dx = px - qx
    dy = py - qy
    dz = pz - qz
    dist = dx * dx + dy * dy + dz * dz
    return u, v, w, dist


def _tc_body(pts_ref, ftr_ref, idx_ref, rd_ref, ri_ref):
    j = pl.program_id(1)

    gid = j * BF + lax.broadcasted_iota(jnp.int32, (RS, BF), 1)
    first = j == 0

    def strip(r, carry):
        sl = pl.ds(r * RS, RS)
        px = pts_ref[sl, 0:1]
        py = pts_ref[sl, 1:2]
        pz = pts_ref[sl, 2:3]
        _, _, _, dist = _closest(
            px, py, pz,
            ftr_ref[0:1, :], ftr_ref[1:2, :], ftr_ref[2:3, :],
            ftr_ref[3:4, :], ftr_ref[4:5, :], ftr_ref[5:6, :],
            ftr_ref[6:7, :], ftr_ref[7:8, :], ftr_ref[8:9, :])

        prev_d = rd_ref[sl, :]
        better = first | (dist < prev_d)
        rd_ref[sl, :] = jnp.where(better, dist, prev_d)
        ri_ref[sl, :] = jnp.where(better, gid, ri_ref[sl, :])
        return carry

    lax.fori_loop(0, BP // RS, strip, 0, unroll=32)

    @pl.when(j == NJ - 1)
    def _():
        rd = rd_ref[...]
        ri = ri_ref[...]
        m = jnp.min(rd, axis=1, keepdims=True)
        ismin = rd == m
        cand = jnp.where(ismin, ri, 2 ** 30)
        idx_ref[...] = jnp.min(cand, axis=1)


def _pairwise_tc(points, ftr):
    return pl.pallas_call(
        _tc_body,
        grid=(NI, NJ),
        in_specs=[
            pl.BlockSpec((BP, 3), lambda i, j: (i, 0)),
            pl.BlockSpec((9, BF), lambda i, j: (0, j)),
        ],
        out_specs=pl.BlockSpec((BP,), lambda i, j: (i,)),
        out_shape=jax.ShapeDtypeStruct((P,), jnp.int32),
        scratch_shapes=[
            pltpu.VMEM((BP, BF), jnp.float32),
            pltpu.VMEM((BP, BF), jnp.int32),
        ],
        compiler_params=pltpu.CompilerParams(
            dimension_semantics=("parallel", "arbitrary"),
        ),
    )(points, ftr)


def _weights_body(pts_ref, fw_ref, w1_ref, w2_ref, w3_ref):
    u, v, w, _ = _closest(
        pts_ref[0:1, :], pts_ref[1:2, :], pts_ref[2:3, :],
        fw_ref[0:1, :], fw_ref[1:2, :], fw_ref[2:3, :],
        fw_ref[3:4, :], fw_ref[4:5, :], fw_ref[5:6, :],
        fw_ref[6:7, :], fw_ref[7:8, :], fw_ref[8:9, :])
    w1_ref[...] = u[0]
    w2_ref[...] = v[0]
    w3_ref[...] = w[0]


def _weights_tc(pts_t, fwin):
    out_shapes = (
        jax.ShapeDtypeStruct((P,), jnp.float32),
        jax.ShapeDtypeStruct((P,), jnp.float32),
        jax.ShapeDtypeStruct((P,), jnp.float32),
    )
    return pl.pallas_call(
        _weights_body,
        grid=(P // BW,),
        in_specs=[
            pl.BlockSpec((3, BW), lambda i: (0, i)),
            pl.BlockSpec((9, BW), lambda i: (0, i)),
        ],
        out_specs=(
            pl.BlockSpec((BW,), lambda i: (i,)),
            pl.BlockSpec((BW,), lambda i: (i,)),
            pl.BlockSpec((BW,), lambda i: (i,)),
        ),
        out_shape=out_shapes,
    )(pts_t, fwin)


_NW = 32
_BG = 3 * F
_BPW = _BG // _NW


def _sc_gather(table, idx_flat):
    mesh = plsc.VectorSubcoreMesh(core_axis_name="c", subcore_axis_name="s")

    @functools.partial(
        pl.kernel,
        mesh=mesh,
        out_type=jax.ShapeDtypeStruct((_BG, 128), jnp.float32),
        scratch_types=[
            pltpu.VMEM((_BPW,), jnp.int32),
            pltpu.VMEM((_BPW, 128), jnp.float32),
            pltpu.SemaphoreType.DMA,
        ],
    )
    def gk(table_hbm, idx_hbm, out_hbm, idx_v, rows_v, sem):
        wid = lax.axis_index("s") * 2 + lax.axis_index("c")
        base = wid * _BPW
        pltpu.sync_copy(idx_hbm.at[pl.ds(base, _BPW)], idx_v)
        pltpu.async_copy(table_hbm.at[idx_v], rows_v, sem).wait()
        pltpu.sync_copy(rows_v, out_hbm.at[pl.ds(base, _BPW)])

    return gk(table, idx_flat)


def kernel(points, vertices, faces):
    vert_pad = jnp.pad(vertices, ((0, 0), (0, 125)))
    idx_flat = faces.T.reshape(-1)
    g = _sc_gather(vert_pad, idx_flat)
    ftr = g.reshape(3, F, 128)[:, :, :3].transpose(0, 2, 1).reshape(9, F)
    idx = _pairwise_tc(points, ftr)
    idx2 = jnp.concatenate([idx, idx + F, idx + 2 * F])
    g2 = _sc_gather(g, idx2)
    fwin = g2.reshape(3, P, 128)[:, :, :3].transpose(0, 2, 1).reshape(9, P)
    w1, w2, w3 = _weights_tc(points.T, fwin)
    return idx, w1, w2, w3


# speedup vs baseline: 1.0900x; 1.0282x over previous
import functools

import jax
import jax.numpy as jnp
from jax import lax
from jax.experimental import pallas as pl
from jax.experimental.pallas import tpu as pltpu
from jax.experimental.pallas import tpu_sc as plsc

P = 8192
F = 8192
V = 4096
BP = 512
BF = 256
RS = 8
NI = P // BP
NJ = F // BF
BW = 1024


def _safe_div(num, den):
    den_s = jnp.where(jnp.abs(den) > 1e-12, den, jnp.ones_like(den))
    return num / den_s


def _closest(px, py, pz, ax, ay, az, bx, by, bz, cx, cy, cz):
    abx = bx - ax
    aby = by - ay
    abz = bz - az
    acx = cx - ax
    acy = cy - ay
    acz = cz - az

    apx = px - ax
    apy = py - ay
    apz = pz - az
    d1 = abx * apx + aby * apy + abz * apz
    d2 = acx * apx + acy * apy + acz * apz
    bpx = px - bx
    bpy = py - by
    bpz = pz - bz
    d3 = abx * bpx + aby * bpy + abz * bpz
    d4 = acx * bpx + acy * bpy + acz * bpz
    cpx = px - cx
    cpy = py - cy
    cpz = pz - cz
    d5 = abx * cpx + aby * cpy + abz * cpz
    d6 = acx * cpx + acy * cpy + acz * cpz

    vc = d1 * d4 - d3 * d2
    vb = d5 * d2 - d1 * d6
    va = d3 * d6 - d5 * d4
    v_ab = _safe_div(d1, d1 - d3)
    w_ac = _safe_div(d2, d2 - d6)
    w_bc = _safe_div(d4 - d3, (d4 - d3) + (d5 - d6))
    denom = _safe_div(jnp.ones_like(va), va + vb + vc)
    v_in = vb * denom
    w_in = vc * denom
    cond_a = (d1 <= 0.0) & (d2 <= 0.0)
    cond_b = (d3 >= 0.0) & (d4 <= d3)
    cond_ab = (vc <= 0.0) & (d1 >= 0.0) & (d3 <= 0.0)
    cond_c = (d6 >= 0.0) & (d5 <= d6)
    cond_ac = (vb <= 0.0) & (d2 >= 0.0) & (d6 <= 0.0)
    cond_bc = (va <= 0.0) & ((d4 - d3) >= 0.0) & ((d5 - d6) >= 0.0)

    u = 1.0 - v_in - w_in
    v = v_in
    w = w_in
    u = jnp.where(cond_bc, 0.0, u)
    v = jnp.where(cond_bc, 1.0 - w_bc, v)
    w = jnp.where(cond_bc, w_bc, w)
    u = jnp.where(cond_ac, 1.0 - w_ac, u)
    v = jnp.where(cond_ac, 0.0, v)
    w = jnp.where(cond_ac, w_ac, w)
    u = jnp.where(cond_c, 0.0, u)
    v = jnp.where(cond_c, 0.0, v)
    w = jnp.where(cond_c, 1.0, w)
    u = jnp.where(cond_ab, 1.0 - v_ab, u)
    v = jnp.where(cond_ab, v_ab, v)
    w = jnp.where(cond_ab, 0.0, w)
    u = jnp.where(cond_b, 0.0, u)
    v = jnp.where(cond_b, 1.0, v)
    w = jnp.where(cond_b, 0.0, w)
    u = jnp.where(cond_a, 1.0, u)
    v = jnp.where(cond_a, 0.0, v)
    w = jnp.where(cond_a, 0.0, w)

    qx = u * ax + v * bx + w * cx
    qy = u * ay + v * by + w * cy
    qz = u * az + v * bz + w * cz
    dx = px - qx
    dy = py - qy
    dz = pz - qz
    dist = dx * dx + dy * dy + dz * dz
    return u, v, w, dist


def _tc_body(pts_ref, ftr_ref, idx_ref, rd_ref, ri_ref):
    j = pl.program_id(1)

    gid = j * BF + lax.broadcasted_iota(jnp.int32, (RS, BF), 1)
    first = j == 0

    def strip(r, carry):
        sl = pl.ds(r * RS, RS)
        px = pts_ref[sl, 0:1]
        py = pts_ref[sl, 1:2]
        pz = pts_ref[sl, 2:3]
        _, _, _, dist = _closest(
            px, py, pz,
            ftr_ref[0:1, :], ftr_ref[1:2, :], ftr_ref[2:3, :],
            ftr_ref[3:4, :], ftr_ref[4:5, :], ftr_ref[5:6, :],
            ftr_ref[6:7, :], ftr_ref[7:8, :], ftr_ref[8:9, :])

        prev_d = rd_ref[sl, :]
        better = first | (dist < prev_d)
        rd_ref[sl, :] = jnp.where(better, dist, prev_d)
        ri_ref[sl, :] = jnp.where(better, gid, ri_ref[sl, :])
        return carry

    lax.fori_loop(0, BP // RS, strip, 0, unroll=32)

    @pl.when(j == NJ - 1)
    def _():
        rd = rd_ref[...]
        ri = ri_ref[...]
        m = jnp.min(rd, axis=1, keepdims=True)
        ismin = rd == m
        cand = jnp.where(ismin, ri, 2 ** 30)
        idx_ref[...] = jnp.min(cand, axis=1)


def _pairwise_tc(points, ftr):
    return pl.pallas_call(
        _tc_body,
        grid=(NI, NJ),
        in_specs=[
            pl.BlockSpec((BP, 3), lambda i, j: (i, 0)),
            pl.BlockSpec((9, BF), lambda i, j: (0, j)),
        ],
        out_specs=pl.BlockSpec((BP,), lambda i, j: (i,)),
        out_shape=jax.ShapeDtypeStruct((P,), jnp.int32),
        scratch_shapes=[
            pltpu.VMEM((BP, BF), jnp.float32),
            pltpu.VMEM((BP, BF), jnp.int32),
        ],
        compiler_params=pltpu.CompilerParams(
            dimension_semantics=("parallel", "arbitrary"),
        ),
    )(points, ftr)


def _weights_body(pts_ref, fw_ref, w1_ref, w2_ref, w3_ref):
    u, v, w, _ = _closest(
        pts_ref[0:1, :], pts_ref[1:2, :], pts_ref[2:3, :],
        fw_ref[0:1, :], fw_ref[1:2, :], fw_ref[2:3, :],
        fw_ref[3:4, :], fw_ref[4:5, :], fw_ref[5:6, :],
        fw_ref[6:7, :], fw_ref[7:8, :], fw_ref[8:9, :])
    w1_ref[...] = u[0]
    w2_ref[...] = v[0]
    w3_ref[...] = w[0]


def _weights_tc(pts_t, fwin):
    out_shapes = (
        jax.ShapeDtypeStruct((P,), jnp.float32),
        jax.ShapeDtypeStruct((P,), jnp.float32),
        jax.ShapeDtypeStruct((P,), jnp.float32),
    )
    return pl.pallas_call(
        _weights_body,
        grid=(P // BW,),
        in_specs=[
            pl.BlockSpec((3, BW), lambda i: (0, i)),
            pl.BlockSpec((9, BW), lambda i: (0, i)),
        ],
        out_specs=(
            pl.BlockSpec((BW,), lambda i: (i,)),
            pl.BlockSpec((BW,), lambda i: (i,)),
            pl.BlockSpec((BW,), lambda i: (i,)),
        ),
        out_shape=out_shapes,
    )(pts_t, fwin)


_NW = 32
_BG = 3 * F
_BPW = _BG // _NW


def _sc_gather(table, idx_flat):
    mesh = plsc.VectorSubcoreMesh(core_axis_name="c", subcore_axis_name="s")

    @functools.partial(
        pl.kernel,
        mesh=mesh,
        out_type=jax.ShapeDtypeStruct((_BG, 128), jnp.float32),
        scratch_types=[
            pltpu.VMEM((_BPW,), jnp.int32),
            pltpu.VMEM((_BPW, 128), jnp.float32),
            pltpu.SemaphoreType.DMA,
        ],
    )
    def gk(table_hbm, idx_hbm, out_hbm, idx_v, rows_v, sem):
        wid = lax.axis_index("s") * 2 + lax.axis_index("c")
        base = wid * _BPW
        pltpu.sync_copy(idx_hbm.at[pl.ds(base, _BPW)], idx_v)
        pltpu.async_copy(table_hbm.at[idx_v], rows_v, sem).wait()
        pltpu.sync_copy(rows_v, out_hbm.at[pl.ds(base, _BPW)])

    return gk(table, idx_flat)


def kernel(points, vertices, faces):
    vert_pad = jnp.pad(vertices, ((0, 0), (0, 125)))
    idx_flat = faces.T.reshape(-1)
    g = _sc_gather(vert_pad, idx_flat)
    ftr = g.reshape(3, F, 128)[:, :, :3].transpose(0, 2, 1).reshape(9, F)
    idx = _pairwise_tc(points, ftr)
    idx2 = jnp.concatenate([idx, idx + F, idx + 2 * F])
    g2 = _sc_gather(g, idx2)
    fwin = g2.reshape(3, P, 128)[:, :, :3].transpose(0, 2, 1).reshape(9, P)
    w1, w2, w3 = _weights_tc(points.T, fwin)
    return idx, w1, w2, w3


# speedup vs baseline: 1.0989x; 1.0082x over previous
import functools

import jax
import jax.numpy as jnp
from jax import lax
from jax.experimental import pallas as pl
from jax.experimental.pallas import tpu as pltpu
from jax.experimental.pallas import tpu_sc as plsc

P = 8192
F = 8192
V = 4096
BP = 1024
BF = 256
RS = 8
NI = P // BP
NJ = F // BF
BW = 1024


def _safe_div(num, den):
    den_s = jnp.where(jnp.abs(den) > 1e-12, den, jnp.ones_like(den))
    return num / den_s


def _closest(px, py, pz, ax, ay, az, bx, by, bz, cx, cy, cz):
    abx = bx - ax
    aby = by - ay
    abz = bz - az
    acx = cx - ax
    acy = cy - ay
    acz = cz - az

    apx = px - ax
    apy = py - ay
    apz = pz - az
    d1 = abx * apx + aby * apy + abz * apz
    d2 = acx * apx + acy * apy + acz * apz
    bpx = px - bx
    bpy = py - by
    bpz = pz - bz
    d3 = abx * bpx + aby * bpy + abz * bpz
    d4 = acx * bpx + acy * bpy + acz * bpz
    cpx = px - cx
    cpy = py - cy
    cpz = pz - cz
    d5 = abx * cpx + aby * cpy + abz * cpz
    d6 = acx * cpx + acy * cpy + acz * cpz

    vc = d1 * d4 - d3 * d2
    vb = d5 * d2 - d1 * d6
    va = d3 * d6 - d5 * d4
    v_ab = _safe_div(d1, d1 - d3)
    w_ac = _safe_div(d2, d2 - d6)
    w_bc = _safe_div(d4 - d3, (d4 - d3) + (d5 - d6))
    denom = _safe_div(jnp.ones_like(va), va + vb + vc)
    v_in = vb * denom
    w_in = vc * denom
    cond_a = (d1 <= 0.0) & (d2 <= 0.0)
    cond_b = (d3 >= 0.0) & (d4 <= d3)
    cond_ab = (vc <= 0.0) & (d1 >= 0.0) & (d3 <= 0.0)
    cond_c = (d6 >= 0.0) & (d5 <= d6)
    cond_ac = (vb <= 0.0) & (d2 >= 0.0) & (d6 <= 0.0)
    cond_bc = (va <= 0.0) & ((d4 - d3) >= 0.0) & ((d5 - d6) >= 0.0)

    u = 1.0 - v_in - w_in
    v = v_in
    w = w_in
    u = jnp.where(cond_bc, 0.0, u)
    v = jnp.where(cond_bc, 1.0 - w_bc, v)
    w = jnp.where(cond_bc, w_bc, w)
    u = jnp.where(cond_ac, 1.0 - w_ac, u)
    v = jnp.where(cond_ac, 0.0, v)
    w = jnp.where(cond_ac, w_ac, w)
    u = jnp.where(cond_c, 0.0, u)
    v = jnp.where(cond_c, 0.0, v)
    w = jnp.where(cond_c, 1.0, w)
    u = jnp.where(cond_ab, 1.0 - v_ab, u)
    v = jnp.where(cond_ab, v_ab, v)
    w = jnp.where(cond_ab, 0.0, w)
    u = jnp.where(cond_b, 0.0, u)
    v = jnp.where(cond_b, 1.0, v)
    w = jnp.where(cond_b, 0.0, w)
    u = jnp.where(cond_a, 1.0, u)
    v = jnp.where(cond_a, 0.0, v)
    w = jnp.where(cond_a, 0.0, w)

    qx = u * ax + v * bx + w * cx
    qy = u * ay + v * by + w * cy
    qz = u * az + v * bz + w * cz
    dx = px - qx
    dy = py - qy
    dz = pz - qz
    dist = dx * dx + dy * dy + dz * dz
    return u, v, w, dist


def _tc_body(pts_ref, ftr_ref, idx_ref, rd_ref, ri_ref):
    j = pl.program_id(1)

    gid = j * BF + lax.broadcasted_iota(jnp.int32, (RS, BF), 1)
    first = j == 0

    def strip(r, carry):
        sl = pl.ds(r * RS, RS)
        px = pts_ref[sl, 0:1]
        py = pts_ref[sl, 1:2]
        pz = pts_ref[sl, 2:3]
        _, _, _, dist = _closest(
            px, py, pz,
            ftr_ref[0:1, :], ftr_ref[1:2, :], ftr_ref[2:3, :],
            ftr_ref[3:4, :], ftr_ref[4:5, :], ftr_ref[5:6, :],
            ftr_ref[6:7, :], ftr_ref[7:8, :], ftr_ref[8:9, :])

        prev_d = rd_ref[sl, :]
        better = first | (dist < prev_d)
        rd_ref[sl, :] = jnp.where(better, dist, prev_d)
        ri_ref[sl, :] = jnp.where(better, gid, ri_ref[sl, :])
        return carry

    lax.fori_loop(0, BP // RS, strip, 0, unroll=32)

    @pl.when(j == NJ - 1)
    def _():
        rd = rd_ref[...]
        ri = ri_ref[...]
        m = jnp.min(rd, axis=1, keepdims=True)
        ismin = rd == m
        cand = jnp.where(ismin, ri, 2 ** 30)
        idx_ref[...] = jnp.min(cand, axis=1)


def _pairwise_tc(points, ftr):
    return pl.pallas_call(
        _tc_body,
        grid=(NI, NJ),
        in_specs=[
            pl.BlockSpec((BP, 3), lambda i, j: (i, 0)),
            pl.BlockSpec((9, BF), lambda i, j: (0, j)),
        ],
        out_specs=pl.BlockSpec((BP,), lambda i, j: (i,)),
        out_shape=jax.ShapeDtypeStruct((P,), jnp.int32),
        scratch_shapes=[
            pltpu.VMEM((BP, BF), jnp.float32),
            pltpu.VMEM((BP, BF), jnp.int32),
        ],
        compiler_params=pltpu.CompilerParams(
            dimension_semantics=("parallel", "arbitrary"),
        ),
    )(points, ftr)


def _weights_body(pts_ref, fw_ref, w1_ref, w2_ref, w3_ref):
    u, v, w, _ = _closest(
        pts_ref[0:1, :], pts_ref[1:2, :], pts_ref[2:3, :],
        fw_ref[0:1, :], fw_ref[1:2, :], fw_ref[2:3, :],
        fw_ref[3:4, :], fw_ref[4:5, :], fw_ref[5:6, :],
        fw_ref[6:7, :], fw_ref[7:8, :], fw_ref[8:9, :])
    w1_ref[...] = u[0]
    w2_ref[...] = v[0]
    w3_ref[...] = w[0]


def _weights_tc(pts_t, fwin):
    out_shapes = (
        jax.ShapeDtypeStruct((P,), jnp.float32),
        jax.ShapeDtypeStruct((P,), jnp.float32),
        jax.ShapeDtypeStruct((P,), jnp.float32),
    )
    return pl.pallas_call(
        _weights_body,
        grid=(P // BW,),
        in_specs=[
            pl.BlockSpec((3, BW), lambda i: (0, i)),
            pl.BlockSpec((9, BW), lambda i: (0, i)),
        ],
        out_specs=(
            pl.BlockSpec((BW,), lambda i: (i,)),
            pl.BlockSpec((BW,), lambda i: (i,)),
            pl.BlockSpec((BW,), lambda i: (i,)),
        ),
        out_shape=out_shapes,
    )(pts_t, fwin)


_NW = 32
_BG = 3 * F
_BPW = _BG // _NW


def _sc_gather(table, idx_flat):
    mesh = plsc.VectorSubcoreMesh(core_axis_name="c", subcore_axis_name="s")

    @functools.partial(
        pl.kernel,
        mesh=mesh,
        out_type=jax.ShapeDtypeStruct((_BG, 128), jnp.float32),
        scratch_types=[
            pltpu.VMEM((_BPW,), jnp.int32),
            pltpu.VMEM((_BPW, 128), jnp.float32),
            pltpu.SemaphoreType.DMA,
        ],
    )
    def gk(table_hbm, idx_hbm, out_hbm, idx_v, rows_v, sem):
        wid = lax.axis_index("s") * 2 + lax.axis_index("c")
        base = wid * _BPW
        pltpu.sync_copy(idx_hbm.at[pl.ds(base, _BPW)], idx_v)
        pltpu.async_copy(table_hbm.at[idx_v], rows_v, sem).wait()
        pltpu.sync_copy(rows_v, out_hbm.at[pl.ds(base, _BPW)])

    return gk(table, idx_flat)


def kernel(points, vertices, faces):
    vert_pad = jnp.pad(vertices, ((0, 0), (0, 125)))
    idx_flat = faces.T.reshape(-1)
    g = _sc_gather(vert_pad, idx_flat)
    ftr = g.reshape(3, F, 128)[:, :, :3].transpose(0, 2, 1).reshape(9, F)
    idx = _pairwise_tc(points, ftr)
    idx2 = jnp.concatenate([idx, idx + F, idx + 2 * F])
    g2 = _sc_gather(g, idx2)
    fwin = g2.reshape(3, P, 128)[:, :, :3].transpose(0, 2, 1).reshape(9, P)
    w1, w2, w3 = _weights_tc(points.T, fwin)
    return idx, w1, w2, w3
